# Initial kernel scaffold; baseline (speedup 1.0000x reference)
import jax
import jax.numpy as jnp
from jax.experimental import pallas as pl


def kernel(x):
    raise NotImplementedError("write your pallas kernel here")


# speedup vs baseline: 40.8398x; 40.8398x over previous
import functools

import jax
import jax.numpy as jnp
from jax import lax
from jax.experimental import pallas as pl
from jax.experimental.pallas import tpu as pltpu
from jax.experimental.pallas import tpu_sc as plsc

N = 32768
K = 64
ROWS = 2048
L = 16
NW = 32
RPW = ROWS // NW
CAP = 512
TRIG = 352
CHUNK = 8
NCHUNK = N // (CHUNK * L)
NEG = float("-inf")
POS = float("inf")


def _scalar0(v):
    return v[0]


def _vsort_desc(v):
    return plsc.sort_key_val(v, v, descending=True)[0]


def _rev(v):
    return lax.rev(v, (0,))


def _bitonic_desc(vs):
    m = len(vs)
    if m == 1:
        return [_vsort_desc(vs[0])]
    h = m // 2
    hi = [jnp.maximum(vs[k], vs[k + h]) for k in range(h)]
    lo = [jnp.minimum(vs[k], vs[k + h]) for k in range(h)]
    return _bitonic_desc(hi) + _bitonic_desc(lo)


def _merge_desc(a, b, keep):
    m = len(a)
    br = [_rev(x) for x in reversed(b)]
    hi = [jnp.maximum(a[k], br[k]) for k in range(m)]
    out = _bitonic_desc(hi)
    if keep > m:
        lo = [jnp.minimum(a[k], br[k]) for k in range(m)]
        out = out + _bitonic_desc(lo)
    return out[:keep]


def _top4_desc(vs):
    lists = [[_vsort_desc(v)] for v in vs]
    while len(lists) > 1:
        nxt = []
        for i in range(0, len(lists), 2):
            m = len(lists[i])
            nxt.append(_merge_desc(lists[i], lists[i + 1], min(2 * m, 4)))
        lists = nxt
    return lists[0]


def _shrink(buf, th_ref, c):
    iota = lax.iota(jnp.int32, L)
    ninf = jnp.full((L,), NEG, jnp.float32)
    his = []
    los = []
    for k in range(CAP // L):
        v = buf[pl.ds(16 * k, L)]
        valid = (iota + (16 * k)) < c
        his.append(jnp.where(valid, v, ninf))
        los.append(jnp.where(valid, -v, ninf))
    toph = _top4_desc(his)
    topl = _top4_desc(los)
    thi = jnp.min(toph[3])
    tlo = -jnp.min(topl[3])
    for k in range(4):
        buf[pl.ds(16 * k, L)] = toph[k]
    for k in range(4):
        buf[pl.ds(64 + 16 * k, L)] = -topl[k]
    th_ref[0] = thi
    th_ref[1] = tlo


def _row_topk(inbuf, row_base, buf, th_ref, out_row_ref):
    th_ref[0] = jnp.float32(NEG)
    th_ref[1] = jnp.float32(POS)

    def chunk_body(i, c_vec):
        thi = th_ref[0]
        tlo = th_ref[1]
        base = row_base + i * (CHUNK * L)
        c = c_vec
        for u in range(CHUNK):
            v = inbuf[pl.ds(base + u * L, L)]
            m = (v > thi) | (v < tlo)
            mi = m.astype(jnp.int32)
            idx = c + plsc.cumsum(mi) - mi
            plsc.store_scatter(buf, [idx], v, mask=m)
            c = c + plsc.all_reduce_population_count(m)
        cs = _scalar0(c)
        full = cs > TRIG
        pl.when(full)(lambda: _shrink(buf, th_ref, cs))
        return jnp.where(full, jnp.full((L,), 128, jnp.int32), c)

    c_fin = lax.fori_loop(0, NCHUNK, chunk_body, jnp.zeros((L,), jnp.int32))
    _shrink(buf, th_ref, _scalar0(c_fin))
    for k in range(8):
        out_row_ref[pl.ds(16 * k, L)] = buf[pl.ds(16 * k, L)]


@functools.cache
def _build_kernel():
    @functools.partial(
        pl.kernel,
        out_type=jax.ShapeDtypeStruct((ROWS, 2 * K), jnp.float32),
        mesh=plsc.VectorSubcoreMesh(
            core_axis_name="c", subcore_axis_name="s",
            num_cores=2, num_subcores=16,
        ),
        compiler_params=pltpu.CompilerParams(needs_layout_passes=False),
        scratch_types=[
            pltpu.VMEM((2 * N,), jnp.float32),
            pltpu.VMEM((CAP,), jnp.float32),
            pltpu.VMEM((RPW, 2 * K), jnp.float32),
            pltpu.SMEM((2,), jnp.float32),
            pltpu.SemaphoreType.DMA((2,)),
        ],
    )
    def _minmax_sc(x_hbm, out_hbm, inbuf, buf, outstage, th_ref, sem):
        wid = lax.axis_index("s") * 2 + lax.axis_index("c")
        base_row = wid * RPW

        pltpu.make_async_copy(
            x_hbm.at[base_row], inbuf.at[pl.ds(0, N)], sem.at[0]
        ).start()

        def row_body(r, _):
            slot = lax.rem(r, 2)
            nslot = 1 - slot

            @pl.when(r < RPW - 1)
            def _prefetch():
                pltpu.make_async_copy(
                    x_hbm.at[base_row + r + 1],
                    inbuf.at[pl.ds(nslot * N, N)],
                    sem.at[nslot],
                ).start()

            pltpu.make_async_copy(
                x_hbm.at[base_row + r],
                inbuf.at[pl.ds(slot * N, N)],
                sem.at[slot],
            ).wait()
            _row_topk(inbuf, slot * N, buf, th_ref, outstage.at[r])
            return 0

        lax.fori_loop(0, RPW, row_body, 0)
        pltpu.sync_copy(outstage, out_hbm.at[pl.ds(base_row, RPW)])

    return _minmax_sc


def kernel(x):
    out = _build_kernel()(x.reshape(ROWS, N))
    return out.reshape(x.shape[0], x.shape[1], 2 * K)


# speedup vs baseline: 44.3698x; 1.0864x over previous
import functools

import jax
import jax.numpy as jnp
from jax import lax
from jax.experimental import pallas as pl
from jax.experimental.pallas import tpu as pltpu
from jax.experimental.pallas import tpu_sc as plsc

N = 32768
K = 64
ROWS = 2048
L = 16
NW = 32
RPW = ROWS // NW
CAP = 512
TRIG = 352
CLAMP = 368
CHUNK = 8
NCHUNK = N // (CHUNK * L)
NEG = float("-inf")
POS = float("inf")


def _vsort_desc(v):
    return plsc.sort_key_val(v, v, descending=True)[0]


def _rev(v):
    return lax.rev(v, (0,))


def _bitonic_desc(vs):
    m = len(vs)
    if m == 1:
        return [_vsort_desc(vs[0])]
    h = m // 2
    hi = [jnp.maximum(vs[k], vs[k + h]) for k in range(h)]
    lo = [jnp.minimum(vs[k], vs[k + h]) for k in range(h)]
    return _bitonic_desc(hi) + _bitonic_desc(lo)


def _merge_desc(a, b, keep):
    m = len(a)
    br = [_rev(x) for x in reversed(b)]
    hi = [jnp.maximum(a[k], br[k]) for k in range(m)]
    out = _bitonic_desc(hi)
    if keep > m:
        lo = [jnp.minimum(a[k], br[k]) for k in range(m)]
        out = out + _bitonic_desc(lo)
    return out[:keep]


def _top4_desc(vs):
    lists = [[_vsort_desc(v)] for v in vs]
    while len(lists) > 1:
        nxt = []
        for i in range(0, len(lists), 2):
            m = len(lists[i])
            nxt.append(_merge_desc(lists[i], lists[i + 1], min(2 * m, 4)))
        lists = nxt
    return lists[0]


def _shrink(buf, th_ref, c):
    iota = lax.iota(jnp.int32, L)
    ninf = jnp.full((L,), NEG, jnp.float32)
    his = []
    los = []
    for k in range(CAP // L):
        v = buf[pl.ds(16 * k, L)]
        valid = (iota + (16 * k)) < c
        his.append(jnp.where(valid, v, ninf))
        los.append(jnp.where(valid, -v, ninf))
    toph = _top4_desc(his)
    topl = _top4_desc(los)
    thi = jnp.min(toph[3])
    tlo = -jnp.min(topl[3])
    for k in range(4):
        buf[pl.ds(16 * k, L)] = toph[k]
    for k in range(4):
        buf[pl.ds(64 + 16 * k, L)] = -topl[k]
    th_ref[0] = thi
    th_ref[1] = tlo


def _row_topk(inbuf, row_base, buf, th_ref, out_row_ref):
    warm_thi = th_ref[0]
    warm_tlo = th_ref[1]
    ones = jnp.full((L,), 1, jnp.int32)

    def _fast_body(i, cm1):
        base = row_base + i * (CHUNK * L)
        vs = [inbuf[pl.ds(base + u * L, L)] for u in range(CHUNK)]
        ms = [(v > warm_thi) | (v < warm_tlo) for v in vs]
        pops = [plsc.all_reduce_population_count(m) for m in ms]
        bases = [cm1]
        for u in range(CHUNK):
            bases.append(bases[u] + pops[u])
        for u in range(CHUNK):
            idx = bases[u] + plsc.cumsum(ones, mask=ms[u])
            plsc.store_scatter(buf, [idx], vs[u], mask=ms[u])
        return jnp.minimum(bases[CHUNK], jnp.full((L,), CLAMP - 1, jnp.int32))

    cm1_end = plsc.parallel_loop(
        0, NCHUNK, carry=jnp.full((L,), -1, jnp.int32)
    )(_fast_body)
    cs = cm1_end[0] + 1
    _shrink(buf, th_ref, cs)
    ok = (cs < CLAMP) & (th_ref[0] > warm_thi) & (th_ref[1] < warm_tlo)

    @pl.when(jnp.logical_not(ok))
    def _cold():
        th_ref[0] = jnp.float32(NEG)
        th_ref[1] = jnp.float32(POS)

        def chunk_body(i, cm1):
            thi = th_ref[0]
            tlo = th_ref[1]
            base = row_base + i * (CHUNK * L)
            c = cm1
            for u in range(CHUNK):
                v = inbuf[pl.ds(base + u * L, L)]
                m = (v > thi) | (v < tlo)
                idx = c + plsc.cumsum(ones, mask=m)
                plsc.store_scatter(buf, [idx], v, mask=m)
                c = c + plsc.all_reduce_population_count(m)
            csc = c[0] + 1
            full = csc > TRIG
            pl.when(full)(lambda: _shrink(buf, th_ref, csc))
            return jnp.where(full, jnp.full((L,), 127, jnp.int32), c)

        cm1_fin = lax.fori_loop(
            0, NCHUNK, chunk_body, jnp.full((L,), -1, jnp.int32)
        )
        _shrink(buf, th_ref, cm1_fin[0] + 1)

    for k in range(8):
        out_row_ref[pl.ds(16 * k, L)] = buf[pl.ds(16 * k, L)]


@functools.cache
def _build_kernel():
    @functools.partial(
        pl.kernel,
        out_type=jax.ShapeDtypeStruct((ROWS, 2 * K), jnp.float32),
        mesh=plsc.VectorSubcoreMesh(
            core_axis_name="c", subcore_axis_name="s",
            num_cores=2, num_subcores=16,
        ),
        compiler_params=pltpu.CompilerParams(needs_layout_passes=False),
        scratch_types=[
            pltpu.VMEM((2 * N,), jnp.float32),
            pltpu.VMEM((CAP,), jnp.float32),
            pltpu.VMEM((RPW, 2 * K), jnp.float32),
            pltpu.SMEM((2,), jnp.float32),
            pltpu.SemaphoreType.DMA((2,)),
        ],
    )
    def _minmax_sc(x_hbm, out_hbm, inbuf, buf, outstage, th_ref, sem):
        wid = lax.axis_index("s") * 2 + lax.axis_index("c")
        base_row = wid * RPW

        th_ref[0] = jnp.float32(NEG)
        th_ref[1] = jnp.float32(POS)
        pltpu.make_async_copy(
            x_hbm.at[base_row], inbuf.at[pl.ds(0, N)], sem.at[0]
        ).start()

        def row_body(r, _):
            slot = lax.rem(r, 2)
            nslot = 1 - slot

            @pl.when(r < RPW - 1)
            def _prefetch():
                pltpu.make_async_copy(
                    x_hbm.at[base_row + r + 1],
                    inbuf.at[pl.ds(nslot * N, N)],
                    sem.at[nslot],
                ).start()

            pltpu.make_async_copy(
                x_hbm.at[base_row + r],
                inbuf.at[pl.ds(slot * N, N)],
                sem.at[slot],
            ).wait()
            _row_topk(inbuf, slot * N, buf, th_ref, outstage.at[r])
            return 0

        lax.fori_loop(0, RPW, row_body, 0)
        pltpu.sync_copy(outstage, out_hbm.at[pl.ds(base_row, RPW)])

    return _minmax_sc


def kernel(x):
    out = _build_kernel()(x.reshape(ROWS, N))
    return out.reshape(x.shape[0], x.shape[1], 2 * K)
